# baseline (speedup 1.0000x reference)
import functools

import jax
import jax.numpy as jnp
from jax import lax
from jax.experimental import pallas as pl
from jax.experimental.pallas import tpu as pltpu
from jax.experimental.pallas import tpu_sc as plsc

BATCH = 16384
EMB = 64
NUM_NUMERIC = 12
NUM_STYLES = 50

_NC = 2
_NS = 16
_NW = _NC * _NS
_BPW = BATCH // _NW
_CHUNK = 128
_NCHUNK = _BPW // _CHUNK

_PACK_R = 16384


def _pack_body(in_ref, o_ref):
    x = in_ref[...]
    eye = jnp.eye(EMB, dtype=jnp.float32)
    t = lax.dot_general(x, eye, (((0,), (0,)), ((), ())),
                        preferred_element_type=jnp.float32)
    t4 = t.reshape(t.shape[0] // 128, 2, 64, 64)
    left = t4[:, 0].reshape(-1, 64)
    right = t4[:, 1].reshape(-1, 64)
    o_ref[...] = jnp.concatenate([left, right], axis=-1)


def _pack(tabT):
    rows = tabT.shape[1]
    out_rows = 64 * ((rows + 127) // 128)
    grid = (pl.cdiv(out_rows, _PACK_R),)
    return pl.pallas_call(
        _pack_body,
        grid=grid,
        in_specs=[pl.BlockSpec((EMB, 2 * _PACK_R), lambda i: (0, i))],
        out_specs=pl.BlockSpec((_PACK_R, 128), lambda i: (i, 0)),
        out_shape=jax.ShapeDtypeStruct((out_rows, 128), jnp.float32),
    )(tabT)


def _sc_gather(q3, tab2):
    mesh = plsc.VectorSubcoreMesh(core_axis_name="c", subcore_axis_name="s")

    @functools.partial(
        pl.kernel,
        mesh=mesh,
        out_type=jax.ShapeDtypeStruct((BATCH, 128), jnp.float32),
        scratch_types=[
            pltpu.VMEM((_NCHUNK, _CHUNK), jnp.int32),
            pltpu.VMEM((2, _CHUNK, 128), jnp.float32),
            pltpu.SemaphoreType.DMA,
        ],
    )
    def k(q_hbm, tab_hbm, out_hbm, q_v, buf_v, sem):
        wid = lax.axis_index("s") * _NC + lax.axis_index("c")
        base = wid * _BPW
        pltpu.sync_copy(q_hbm.at[wid], q_v)
        c = [None, None]
        c[0] = pltpu.async_copy(tab_hbm.at[q_v.at[0]], buf_v.at[0], sem)
        for j in range(_NCHUNK):
            b = j % 2
            nb = (j + 1) % 2
            if j + 1 < _NCHUNK:
                c[nb] = pltpu.async_copy(tab_hbm.at[q_v.at[j + 1]],
                                         buf_v.at[nb], sem)
            c[b].wait()
            pltpu.sync_copy(buf_v.at[b],
                            out_hbm.at[pl.ds(base + j * _CHUNK, _CHUNK)])

    return k(q3, tab2)


def _mlp_body(ug_ref, pg_ref, sb_ref, ffT_ref, Wn_ref, bn_ref,
              Ws_ref, bs_ref, W1_ref, b1_ref,
              W2T_ref, b2_ref, W3T_ref, b3_ref, wf_ref, bf_ref, o_ref):
    f32 = jnp.float32
    dT0 = (((0,), (0,)), ((), ()))
    dT1 = (((1,), (1,)), ((), ()))
    ug = ug_ref[...]
    pg = pg_ref[...]
    sb = sb_ref[...].astype(jnp.int32)
    u = jnp.where((sb & 1) == 1, ug[:, EMB:], ug[:, :EMB])
    p = jnp.where((sb & 2) == 2, pg[:, EMB:], pg[:, :EMB])
    ffT = ffT_ref[...]
    nvec = jnp.maximum(
        lax.dot_general(ffT, Wn_ref[...], dT0, preferred_element_type=f32)
        + bn_ref[...], 0.0)
    svec = jnp.maximum(
        lax.dot_general(ffT, Ws_ref[...], dT0, preferred_element_type=f32)
        + bs_ref[...], 0.0)
    comb = jnp.concatenate([u, p, nvec, svec], axis=-1)
    h = jnp.maximum(
        jnp.dot(comb, W1_ref[...], preferred_element_type=f32)
        + b1_ref[...], 0.0)
    x2 = jnp.maximum(
        lax.dot_general(h, W2T_ref[...], dT1, preferred_element_type=f32)
        + b2_ref[...], 0.0)
    x3 = jnp.maximum(
        lax.dot_general(x2, W3T_ref[...], dT1, preferred_element_type=f32)
        + b3_ref[...], 0.0)
    logitT = lax.dot_general(wf_ref[...], x3, dT1,
                             preferred_element_type=f32) + bf_ref[...]
    o_ref[...] = jax.nn.sigmoid(logitT)


def _mlp(ug, pg, sb, ffT, Wn, bn, Ws, bs, W1, b1,
         W2T, b2, W3T, b3, wf_row, bf):
    R = 8192
    grid = (BATCH // R,)

    def rows(i):
        return (i, 0)

    def whole(i):
        return (0, 0)

    row_spec = lambda w: pl.BlockSpec((R, w), rows)
    full_spec = lambda a: pl.BlockSpec(a.shape, whole)

    return pl.pallas_call(
        _mlp_body,
        grid=grid,
        in_specs=[
            row_spec(128), row_spec(128), row_spec(1),
            pl.BlockSpec((62, R), lambda i: (0, i)),
            full_spec(Wn), full_spec(bn), full_spec(Ws), full_spec(bs),
            full_spec(W1), full_spec(b1),
            full_spec(W2T), full_spec(b2),
            full_spec(W3T), full_spec(b3), full_spec(wf_row), full_spec(bf),
        ],
        out_specs=pl.BlockSpec((1, R), lambda i: (0, i)),
        out_shape=jax.ShapeDtypeStruct((1, BATCH), jnp.float32),
    )(ug, pg, sb, ffT, Wn, bn, Ws, bs, W1, b1,
      W2T, b2, W3T, b3, wf_row, bf)


def kernel(user_id, product_id, full_features, user_table, product_table,
           W_num, b_num, W_style, b_style, W1, b1, W2, b2, W3, b3, Wf, bf):
    uid = user_id.astype(jnp.int32)
    pid = product_id.astype(jnp.int32)

    uq = ((uid >> 7) * 64 + (uid & 63)).reshape(_NW, _NCHUNK, _CHUNK)
    pq = ((pid >> 7) * 64 + (pid & 63)).reshape(_NW, _NCHUNK, _CHUNK)
    sb = (((uid >> 6) & 1) | (((pid >> 6) & 1) << 1)).astype(
        jnp.int8).reshape(BATCH, 1)

    ptab2 = _pack(product_table.T)
    pg = _sc_gather(pq, ptab2)
    utab2 = _pack(user_table.T)
    ug = _sc_gather(uq, utab2)

    Wn = jnp.zeros((62, EMB), jnp.float32).at[:NUM_NUMERIC].set(W_num)
    Ws = jnp.zeros((62, EMB), jnp.float32).at[NUM_NUMERIC:].set(W_style)

    res = _mlp(ug, pg, sb, full_features.T,
               Wn, b_num.reshape(1, EMB), Ws, b_style.reshape(1, EMB),
               W1, b1.reshape(1, 128),
               W2.T, b2.reshape(1, 64), W3.T, b3.reshape(1, 32),
               Wf.reshape(1, 32), bf.reshape(1, 1))
    return res.reshape(BATCH, 1)


# speedup vs baseline: 1.0075x; 1.0075x over previous
import functools

import jax
import jax.numpy as jnp
from jax import lax
from jax.experimental import pallas as pl
from jax.experimental.pallas import tpu as pltpu
from jax.experimental.pallas import tpu_sc as plsc

BATCH = 16384
EMB = 64
NUM_NUMERIC = 12
NUM_STYLES = 50

_NC = 2
_NS = 16
_NW = _NC * _NS
_BPW = BATCH // _NW
_CHUNK = 128
_NCHUNK = _BPW // _CHUNK

_PACK_R = 16384


def _pack_body(in_ref, o_ref):
    t = jnp.swapaxes(in_ref[...], 0, 1)
    t4 = t.reshape(t.shape[0] // 128, 2, 64, 64)
    left = t4[:, 0].reshape(-1, 64)
    right = t4[:, 1].reshape(-1, 64)
    o_ref[...] = jnp.concatenate([left, right], axis=-1)


def _pack(tabT):
    rows = tabT.shape[1]
    out_rows = 64 * ((rows + 127) // 128)
    grid = (pl.cdiv(out_rows, _PACK_R),)
    return pl.pallas_call(
        _pack_body,
        grid=grid,
        in_specs=[pl.BlockSpec((EMB, 2 * _PACK_R), lambda i: (0, i))],
        out_specs=pl.BlockSpec((_PACK_R, 128), lambda i: (i, 0)),
        out_shape=jax.ShapeDtypeStruct((out_rows, 128), jnp.float32),
    )(tabT)


def _sc_gather(q3, tab2):
    mesh = plsc.VectorSubcoreMesh(core_axis_name="c", subcore_axis_name="s")

    @functools.partial(
        pl.kernel,
        mesh=mesh,
        out_type=jax.ShapeDtypeStruct((BATCH, 128), jnp.float32),
        scratch_types=[
            pltpu.VMEM((_NCHUNK, _CHUNK), jnp.int32),
            pltpu.VMEM((2, _CHUNK, 128), jnp.float32),
            pltpu.SemaphoreType.DMA,
        ],
    )
    def k(q_hbm, tab_hbm, out_hbm, q_v, buf_v, sem):
        wid = lax.axis_index("s") * _NC + lax.axis_index("c")
        base = wid * _BPW
        pltpu.sync_copy(q_hbm.at[wid], q_v)
        c = [None, None]
        c[0] = pltpu.async_copy(tab_hbm.at[q_v.at[0]], buf_v.at[0], sem)
        for j in range(_NCHUNK):
            b = j % 2
            nb = (j + 1) % 2
            if j + 1 < _NCHUNK:
                c[nb] = pltpu.async_copy(tab_hbm.at[q_v.at[j + 1]],
                                         buf_v.at[nb], sem)
            c[b].wait()
            pltpu.sync_copy(buf_v.at[b],
                            out_hbm.at[pl.ds(base + j * _CHUNK, _CHUNK)])

    return k(q3, tab2)


def _mlp_body(ug_ref, pg_ref, sb_ref, ffT_ref, Wn_ref, bn_ref,
              Ws_ref, bs_ref, W1_ref, b1_ref,
              W2T_ref, b2_ref, W3T_ref, b3_ref, wf_ref, bf_ref, o_ref):
    f32 = jnp.float32
    dT0 = (((0,), (0,)), ((), ()))
    dT1 = (((1,), (1,)), ((), ()))
    ug = ug_ref[...]
    pg = pg_ref[...]
    sb = sb_ref[...].astype(jnp.int32)
    u = jnp.where((sb & 1) == 1, ug[:, EMB:], ug[:, :EMB])
    p = jnp.where((sb & 2) == 2, pg[:, EMB:], pg[:, :EMB])
    ffT = ffT_ref[...]
    nvec = jnp.maximum(
        lax.dot_general(ffT, Wn_ref[...], dT0, preferred_element_type=f32)
        + bn_ref[...], 0.0)
    svec = jnp.maximum(
        lax.dot_general(ffT, Ws_ref[...], dT0, preferred_element_type=f32)
        + bs_ref[...], 0.0)
    comb = jnp.concatenate([u, p, nvec, svec], axis=-1)
    h = jnp.maximum(
        jnp.dot(comb, W1_ref[...], preferred_element_type=f32)
        + b1_ref[...], 0.0)
    x2 = jnp.maximum(
        lax.dot_general(h, W2T_ref[...], dT1, preferred_element_type=f32)
        + b2_ref[...], 0.0)
    x3 = jnp.maximum(
        lax.dot_general(x2, W3T_ref[...], dT1, preferred_element_type=f32)
        + b3_ref[...], 0.0)
    logitT = lax.dot_general(wf_ref[...], x3, dT1,
                             preferred_element_type=f32) + bf_ref[...]
    o_ref[...] = jax.nn.sigmoid(logitT)


def _mlp(ug, pg, sb, ffT, Wn, bn, Ws, bs, W1, b1,
         W2T, b2, W3T, b3, wf_row, bf):
    R = 4096
    grid = (BATCH // R,)

    def rows(i):
        return (i, 0)

    def whole(i):
        return (0, 0)

    row_spec = lambda w: pl.BlockSpec((R, w), rows)
    full_spec = lambda a: pl.BlockSpec(a.shape, whole)

    return pl.pallas_call(
        _mlp_body,
        grid=grid,
        in_specs=[
            row_spec(128), row_spec(128), row_spec(1),
            pl.BlockSpec((62, R), lambda i: (0, i)),
            full_spec(Wn), full_spec(bn), full_spec(Ws), full_spec(bs),
            full_spec(W1), full_spec(b1),
            full_spec(W2T), full_spec(b2),
            full_spec(W3T), full_spec(b3), full_spec(wf_row), full_spec(bf),
        ],
        out_specs=pl.BlockSpec((1, R), lambda i: (0, i)),
        out_shape=jax.ShapeDtypeStruct((1, BATCH), jnp.float32),
    )(ug, pg, sb, ffT, Wn, bn, Ws, bs, W1, b1,
      W2T, b2, W3T, b3, wf_row, bf)


def kernel(user_id, product_id, full_features, user_table, product_table,
           W_num, b_num, W_style, b_style, W1, b1, W2, b2, W3, b3, Wf, bf):
    uid = user_id.astype(jnp.int32)
    pid = product_id.astype(jnp.int32)

    uq = ((uid >> 7) * 64 + (uid & 63)).reshape(_NW, _NCHUNK, _CHUNK)
    pq = ((pid >> 7) * 64 + (pid & 63)).reshape(_NW, _NCHUNK, _CHUNK)
    sb = (((uid >> 6) & 1) | (((pid >> 6) & 1) << 1)).astype(
        jnp.int8).reshape(BATCH, 1)

    ptab2 = _pack(product_table.T)
    pg = _sc_gather(pq, ptab2)
    utab2 = _pack(user_table.T)
    ug = _sc_gather(uq, utab2)

    Wn = jnp.zeros((62, EMB), jnp.float32).at[:NUM_NUMERIC].set(W_num)
    Ws = jnp.zeros((62, EMB), jnp.float32).at[NUM_NUMERIC:].set(W_style)

    res = _mlp(ug, pg, sb, full_features.T,
               Wn, b_num.reshape(1, EMB), Ws, b_style.reshape(1, EMB),
               W1, b1.reshape(1, 128),
               W2.T, b2.reshape(1, 64), W3.T, b3.reshape(1, 32),
               Wf.reshape(1, 32), bf.reshape(1, 1))
    return res.reshape(BATCH, 1)
